# baseline (speedup 1.0000x reference)
import functools

import jax
import jax.numpy as jnp
from jax import lax
from jax.experimental import pallas as pl
from jax.experimental.pallas import tpu as pltpu
from jax.experimental.pallas import tpu_sc as plsc

W = 512
NBUF = 2
SUP = 8
G = 16
L = 16
TAIL = 999936
NFULL = 1953


def _gather_stream(idx, table_t):
    B = idx.shape[0]
    D, V = table_t.shape
    info = plsc.get_sparse_core_info()
    NC, NS = info.num_cores, info.num_subcores
    NW = NC * NS
    OUT_ROWS = B + 128
    base_chunks = NFULL // NW

    @functools.partial(
        pl.kernel,
        mesh=plsc.VectorSubcoreMesh(core_axis_name="c", subcore_axis_name="s"),
        out_type=jax.ShapeDtypeStruct((OUT_ROWS, 128), jnp.float32),
        scratch_types=[
            pltpu.VMEM((B,), jnp.int32),
            pltpu.VMEM((B,), jnp.int32),
            pltpu.VMEM((B,), jnp.int32),
            pltpu.VMEM((NBUF, D, W), jnp.float32),
            pltpu.VMEM((D, V - TAIL), jnp.float32),
            pltpu.VMEM((2, G, 128), jnp.float32),
            pltpu.VMEM((2, G), jnp.int32),
            pltpu.VMEM((L,), jnp.int32),
            pltpu.VMEM((L,), jnp.int32),
            pltpu.SMEM((1,), jnp.int32),
            pltpu.SMEM((1,), jnp.int32),
            pltpu.SMEM((1,), jnp.int32),
            pltpu.SMEM((1,), jnp.int32),
            pltpu.SemaphoreType.DMA,
            pltpu.SemaphoreType.DMA,
            pltpu.SemaphoreType.DMA,
            pltpu.SemaphoreType.DMA,
            pltpu.SemaphoreType.DMA,
        ],
        compiler_params=pltpu.CompilerParams(needs_layout_passes=False),
    )
    def k(table_hbm, idx_hbm, out_hbm, idx_v, hitpos_v, sh_v, slab_v, tail_v,
          rowbuf_v, scatpos_v, extc_v, extp_v, nsm, n2sm, rsm, fsm,
          sem0, sem1, sem2, fsem0, fsem1):
        wid = lax.axis_index("s") * NC + lax.axis_index("c")
        nchunks = jnp.where(wid < 1, jnp.int32(base_chunks + 1),
                            jnp.int32(base_chunks))
        dump = B + wid
        sems = [sem0, sem1, sem2]
        iot = lax.iota(jnp.int32, L)

        def chunk_col(j):
            return (j * NW + wid) * W

        def start_slab(j, slot):
            pltpu.async_copy(
                table_hbm.at[:, pl.ds(chunk_col(j), W)],
                slab_v.at[slot],
                sems[slot],
            )

        def wait_slab(slot):
            pltpu.make_async_copy(
                table_hbm.at[:, pl.ds(0, W)], slab_v.at[slot], sems[slot]
            ).wait()

        for j0 in range(NBUF):
            start_slab(j0, j0)
        pltpu.sync_copy(idx_hbm, idx_v)

        def init_scatpos(p):
            dv = jnp.full((L,), dump, jnp.int32)
            for q in range(G // L):
                scatpos_v[p, pl.ds(q * L, L)] = dv

        init_scatpos(0)
        init_scatpos(1)
        nsm[0] = jnp.int32(0)
        rsm[0] = jnp.int32(0)
        fsm[0] = jnp.int32(0)

        @pl.loop(0, B // L)
        def _l1(i):
            v = idx_v[pl.ds(i * L, L)]
            cid = lax.shift_right_logical(v, 9)
            m = (cid & (NW - 1)) == wid
            kcnt = jnp.sum(m.astype(jnp.int32))
            pos = jnp.full((L,), i * L, jnp.int32) + iot
            plsc.store_compressed(extp_v.at[...], pos, mask=m)
            packed = extp_v[...]
            n = nsm[0]
            plsc.store_scatter(
                hitpos_v.at[...], [jnp.full((L,), n, jnp.int32) + iot],
                packed, mask=iot < kcnt,
            )
            nsm[0] = n + kcnt

        def flush():
            fcount = fsm[0]
            par = fcount & 1

            def issue(p, fsem, other):
                pltpu.async_copy(
                    rowbuf_v.at[p], out_hbm.at[scatpos_v.at[p]], fsem
                )

                @pl.when(fcount >= 1)
                def _():
                    pltpu.make_async_copy(
                        rowbuf_v.at[1 - p], out_hbm.at[pl.ds(0, G)], other
                    ).wait()
                    init_scatpos(1 - p)

            @pl.when(par == 0)
            def _():
                issue(0, fsem0, fsem1)

            @pl.when(par == 1)
            def _():
                issue(1, fsem1, fsem0)

            fsm[0] = fcount + 1

        def process(list_ref, cnt, slab, slab_base, filt_lo, filt_hi):
            ngrp = (cnt + (L - 1)) // L

            @pl.loop(0, ngrp)
            def _grp(g):
                valid = (jnp.full((L,), g * L, jnp.int32) + iot) < cnt
                pv = list_ref[pl.ds(g * L, L)]
                cols = plsc.load_gather(idx_v.at[...], [pv], mask=valid)
                m = valid & (cols >= filt_lo) & (cols < filt_hi)
                k2 = jnp.sum(m.astype(jnp.int32))
                plsc.store_compressed(extc_v.at[...], cols - slab_base, mask=m)
                plsc.store_compressed(extp_v.at[...], pv, mask=m)

                @pl.loop(0, k2)
                def _entry(e):
                    sel = jnp.full((L,), e, jnp.int32)
                    cloc = plsc.load_gather(extc_v.at[...], [sel])
                    pvec = plsc.load_gather(extp_v.at[...], [sel])
                    r2 = rsm[0]
                    parw = jnp.full((L,), fsm[0] & 1, jnp.int32)
                    rfull = jnp.full((L,), r2, jnp.int32)
                    for q in range(D // L):
                        vals = plsc.load_gather(slab.at[...],
                                                [iot + q * L, cloc])
                        plsc.store_scatter(
                            rowbuf_v.at[...], [parw, rfull, iot + q * L], vals
                        )
                    plsc.store_scatter(
                        scatpos_v.at[...], [parw, rfull], pvec, mask=iot == 0
                    )
                    full = r2 + 1 == G

                    @pl.when(full)
                    def _():
                        flush()

                    rsm[0] = jnp.where(full, jnp.int32(0), r2 + 1)

        nsup = (base_chunks + 1 + (SUP - 1)) // SUP

        @pl.loop(0, nsup)
        def _sup(s):
            n = nsm[0]
            n2sm[0] = jnp.int32(0)
            ngrp = (n + (L - 1)) // L

            @pl.loop(0, ngrp)
            def _l15(g):
                valid = (jnp.full((L,), g * L, jnp.int32) + iot) < n
                pv = hitpos_v[pl.ds(g * L, L)]
                cols = plsc.load_gather(idx_v.at[...], [pv], mask=valid)
                jv = lax.shift_right_logical(
                    lax.shift_right_logical(cols, 9) - wid, 5)
                m = valid & (jv >= s * SUP) & (jv < (s + 1) * SUP)
                kcnt = jnp.sum(m.astype(jnp.int32))
                plsc.store_compressed(extp_v.at[...], pv, mask=m)
                packed = extp_v[...]
                n2 = n2sm[0]
                plsc.store_scatter(
                    sh_v.at[...], [jnp.full((L,), n2, jnp.int32) + iot],
                    packed, mask=iot < kcnt,
                )
                n2sm[0] = n2 + kcnt

            n2 = n2sm[0]
            for t in range(SUP):
                j = s * SUP + t
                slot = t % NBUF

                @pl.when(j < nchunks)
                def _():
                    wait_slab(slot)
                    c0 = chunk_col(j)
                    process(sh_v, n2, slab_v.at[slot], c0, c0, c0 + W)

                    @pl.when(j + NBUF < nchunks)
                    def _():
                        start_slab(j + NBUF, slot)

        @pl.when(wid == 1)
        def _():
            pltpu.sync_copy(table_hbm.at[:, pl.ds(TAIL, V - TAIL)], tail_v)
            process(hitpos_v, nsm[0], tail_v, TAIL, TAIL, V)

        flush()
        fcount = fsm[0]
        last = (fcount - 1) & 1

        @pl.when(last == 0)
        def _():
            pltpu.make_async_copy(
                rowbuf_v.at[0], out_hbm.at[pl.ds(0, G)], fsem0
            ).wait()

        @pl.when(last == 1)
        def _():
            pltpu.make_async_copy(
                rowbuf_v.at[1], out_hbm.at[pl.ds(0, G)], fsem1
            ).wait()

    return k(table_t, idx)


def kernel(nodes, ordered_embs):
    idx = nodes.reshape((nodes.shape[0],)).astype(jnp.int32)
    table_t = ordered_embs.T
    out3 = _gather_stream(idx, table_t)
    return out3[: nodes.shape[0], :64]


# speedup vs baseline: 1.1234x; 1.1234x over previous
import functools

import jax
import jax.numpy as jnp
from jax import lax
from jax.experimental import pallas as pl
from jax.experimental.pallas import tpu as pltpu
from jax.experimental.pallas import tpu_sc as plsc

W = 256
NBUF = 4
SUP = 8
G = 16
L = 16
TAIL = 999936
NFULL = 3906


def _gather_stream(idx, table_t):
    B = idx.shape[0]
    D, V = table_t.shape
    info = plsc.get_sparse_core_info()
    NC, NS = info.num_cores, info.num_subcores
    NW = NC * NS
    OUT_ROWS = B + 128
    base_chunks = NFULL // NW

    @functools.partial(
        pl.kernel,
        mesh=plsc.VectorSubcoreMesh(core_axis_name="c", subcore_axis_name="s"),
        out_type=jax.ShapeDtypeStruct((OUT_ROWS, 128), jnp.float32),
        scratch_types=[
            pltpu.VMEM((B,), jnp.int32),
            pltpu.VMEM((B,), jnp.int32),
            pltpu.VMEM((B,), jnp.int32),
            pltpu.VMEM((NBUF, D, W), jnp.float32),
            pltpu.VMEM((D, V - TAIL), jnp.float32),
            pltpu.VMEM((2, G, 128), jnp.float32),
            pltpu.VMEM((2, G), jnp.int32),
            pltpu.VMEM((L,), jnp.int32),
            pltpu.VMEM((L,), jnp.int32),
            pltpu.SMEM((1,), jnp.int32),
            pltpu.SMEM((1,), jnp.int32),
            pltpu.SMEM((1,), jnp.int32),
            pltpu.SMEM((1,), jnp.int32),
            pltpu.SemaphoreType.DMA,
            pltpu.SemaphoreType.DMA,
            pltpu.SemaphoreType.DMA,
            pltpu.SemaphoreType.DMA,
            pltpu.SemaphoreType.DMA,
            pltpu.SemaphoreType.DMA,
        ],
        compiler_params=pltpu.CompilerParams(needs_layout_passes=False),
    )
    def k(table_hbm, idx_hbm, out_hbm, idx_v, hitpos_v, sh_v, slab_v, tail_v,
          rowbuf_v, scatpos_v, extc_v, extp_v, nsm, n2sm, rsm, fsm,
          sem0, sem1, sem2, sem3, fsem0, fsem1):
        wid = lax.axis_index("s") * NC + lax.axis_index("c")
        nchunks = jnp.where(wid < 2, jnp.int32(base_chunks + 1),
                            jnp.int32(base_chunks))
        dump = B + wid
        sems = [sem0, sem1, sem2, sem3]
        iot = lax.iota(jnp.int32, L)

        def chunk_col(j):
            return (j * NW + wid) * W

        def start_slab(j, slot):
            pltpu.async_copy(
                table_hbm.at[:, pl.ds(chunk_col(j), W)],
                slab_v.at[slot],
                sems[slot],
            )

        def wait_slab(slot):
            pltpu.make_async_copy(
                table_hbm.at[:, pl.ds(0, W)], slab_v.at[slot], sems[slot]
            ).wait()

        for j0 in range(NBUF):
            start_slab(j0, j0)
        pltpu.sync_copy(idx_hbm, idx_v)

        def init_scatpos(p):
            dv = jnp.full((L,), dump, jnp.int32)
            for q in range(G // L):
                scatpos_v[p, pl.ds(q * L, L)] = dv

        init_scatpos(0)
        init_scatpos(1)
        nsm[0] = jnp.int32(0)
        rsm[0] = jnp.int32(0)
        fsm[0] = jnp.int32(0)

        @pl.loop(0, B // L)
        def _l1(i):
            v = idx_v[pl.ds(i * L, L)]
            cid = lax.shift_right_logical(v, 8)
            m = (cid & (NW - 1)) == wid
            kcnt = jnp.sum(m.astype(jnp.int32))
            pos = jnp.full((L,), i * L, jnp.int32) + iot
            plsc.store_compressed(extp_v.at[...], pos, mask=m)
            packed = extp_v[...]
            n = nsm[0]
            plsc.store_scatter(
                hitpos_v.at[...], [jnp.full((L,), n, jnp.int32) + iot],
                packed, mask=iot < kcnt,
            )
            nsm[0] = n + kcnt

        def flush():
            fcount = fsm[0]
            par = fcount & 1

            def issue(p, fsem, other):
                pltpu.async_copy(
                    rowbuf_v.at[p], out_hbm.at[scatpos_v.at[p]], fsem
                )

                @pl.when(fcount >= 1)
                def _():
                    pltpu.make_async_copy(
                        rowbuf_v.at[1 - p], out_hbm.at[pl.ds(0, G)], other
                    ).wait()
                    init_scatpos(1 - p)

            @pl.when(par == 0)
            def _():
                issue(0, fsem0, fsem1)

            @pl.when(par == 1)
            def _():
                issue(1, fsem1, fsem0)

            fsm[0] = fcount + 1

        def process(list_ref, cnt, slab, slab_base, filt_lo, filt_hi):
            ngrp = (cnt + (L - 1)) // L

            @pl.loop(0, ngrp)
            def _grp(g):
                valid = (jnp.full((L,), g * L, jnp.int32) + iot) < cnt
                pv = list_ref[pl.ds(g * L, L)]
                cols = plsc.load_gather(idx_v.at[...], [pv], mask=valid)
                m = valid & (cols >= filt_lo) & (cols < filt_hi)
                k2 = jnp.sum(m.astype(jnp.int32))
                plsc.store_compressed(extc_v.at[...], cols - slab_base, mask=m)
                plsc.store_compressed(extp_v.at[...], pv, mask=m)

                @pl.loop(0, k2)
                def _entry(e):
                    sel = jnp.full((L,), e, jnp.int32)
                    cloc = plsc.load_gather(extc_v.at[...], [sel])
                    pvec = plsc.load_gather(extp_v.at[...], [sel])
                    r2 = rsm[0]
                    parw = jnp.full((L,), fsm[0] & 1, jnp.int32)
                    rfull = jnp.full((L,), r2, jnp.int32)
                    for q in range(D // L):
                        vals = plsc.load_gather(slab.at[...],
                                                [iot + q * L, cloc])
                        plsc.store_scatter(
                            rowbuf_v.at[...], [parw, rfull, iot + q * L], vals
                        )
                    plsc.store_scatter(
                        scatpos_v.at[...], [parw, rfull], pvec, mask=iot == 0
                    )
                    full = r2 + 1 == G

                    @pl.when(full)
                    def _():
                        flush()

                    rsm[0] = jnp.where(full, jnp.int32(0), r2 + 1)

        nsup = (base_chunks + 1 + (SUP - 1)) // SUP

        @pl.loop(0, nsup)
        def _sup(s):
            n = nsm[0]
            n2sm[0] = jnp.int32(0)
            ngrp = (n + (L - 1)) // L

            @pl.loop(0, ngrp)
            def _l15(g):
                valid = (jnp.full((L,), g * L, jnp.int32) + iot) < n
                pv = hitpos_v[pl.ds(g * L, L)]
                cols = plsc.load_gather(idx_v.at[...], [pv], mask=valid)
                jv = lax.shift_right_logical(
                    lax.shift_right_logical(cols, 8) - wid, 5)
                m = valid & (jv >= s * SUP) & (jv < (s + 1) * SUP)
                kcnt = jnp.sum(m.astype(jnp.int32))
                plsc.store_compressed(extp_v.at[...], pv, mask=m)
                packed = extp_v[...]
                n2 = n2sm[0]
                plsc.store_scatter(
                    sh_v.at[...], [jnp.full((L,), n2, jnp.int32) + iot],
                    packed, mask=iot < kcnt,
                )
                n2sm[0] = n2 + kcnt

            n2 = n2sm[0]
            for t in range(SUP):
                j = s * SUP + t
                slot = t % NBUF

                @pl.when(j < nchunks)
                def _():
                    wait_slab(slot)
                    c0 = chunk_col(j)
                    process(sh_v, n2, slab_v.at[slot], c0, c0, c0 + W)

                    @pl.when(j + NBUF < nchunks)
                    def _():
                        start_slab(j + NBUF, slot)

        @pl.when(wid == 2)
        def _():
            pltpu.sync_copy(table_hbm.at[:, pl.ds(TAIL, V - TAIL)], tail_v)
            process(hitpos_v, nsm[0], tail_v, TAIL, TAIL, V)

        flush()
        fcount = fsm[0]
        last = (fcount - 1) & 1

        @pl.when(last == 0)
        def _():
            pltpu.make_async_copy(
                rowbuf_v.at[0], out_hbm.at[pl.ds(0, G)], fsem0
            ).wait()

        @pl.when(last == 1)
        def _():
            pltpu.make_async_copy(
                rowbuf_v.at[1], out_hbm.at[pl.ds(0, G)], fsem1
            ).wait()

    return k(table_t, idx)


def kernel(nodes, ordered_embs):
    idx = nodes.reshape((nodes.shape[0],)).astype(jnp.int32)
    table_t = ordered_embs.T
    out3 = _gather_stream(idx, table_t)
    return out3[: nodes.shape[0], :64]


# speedup vs baseline: 1.1317x; 1.0074x over previous
import functools

import jax
import jax.numpy as jnp
from jax import lax
from jax.experimental import pallas as pl
from jax.experimental.pallas import tpu as pltpu
from jax.experimental.pallas import tpu_sc as plsc

W = 128
NBUF = 8
SUP = 16
G = 16
L = 16
TAIL = 999936
NFULL = 7812


def _gather_stream(idx, table_t):
    B = idx.shape[0]
    D, V = table_t.shape
    info = plsc.get_sparse_core_info()
    NC, NS = info.num_cores, info.num_subcores
    NW = NC * NS
    OUT_ROWS = B + 128
    base_chunks = NFULL // NW

    @functools.partial(
        pl.kernel,
        mesh=plsc.VectorSubcoreMesh(core_axis_name="c", subcore_axis_name="s"),
        out_type=jax.ShapeDtypeStruct((OUT_ROWS, 128), jnp.float32),
        scratch_types=[
            pltpu.VMEM((B,), jnp.int32),
            pltpu.VMEM((B,), jnp.int32),
            pltpu.VMEM((B,), jnp.int32),
            pltpu.VMEM((NBUF, D, W), jnp.float32),
            pltpu.VMEM((D, V - TAIL), jnp.float32),
            pltpu.VMEM((2, G, 128), jnp.float32),
            pltpu.VMEM((2, G), jnp.int32),
            pltpu.VMEM((L,), jnp.int32),
            pltpu.VMEM((L,), jnp.int32),
            pltpu.SMEM((1,), jnp.int32),
            pltpu.SMEM((1,), jnp.int32),
            pltpu.SMEM((1,), jnp.int32),
            pltpu.SMEM((1,), jnp.int32),
            pltpu.SemaphoreType.DMA,
            pltpu.SemaphoreType.DMA,
            pltpu.SemaphoreType.DMA,
            pltpu.SemaphoreType.DMA,
            pltpu.SemaphoreType.DMA,
            pltpu.SemaphoreType.DMA,
            pltpu.SemaphoreType.DMA,
            pltpu.SemaphoreType.DMA,
            pltpu.SemaphoreType.DMA,
            pltpu.SemaphoreType.DMA,
        ],
        compiler_params=pltpu.CompilerParams(needs_layout_passes=False),
    )
    def k(table_hbm, idx_hbm, out_hbm, idx_v, hitpos_v, sh_v, slab_v, tail_v,
          rowbuf_v, scatpos_v, extc_v, extp_v, nsm, n2sm, rsm, fsm,
          sem0, sem1, sem2, sem3, sem4, sem5, sem6, sem7, fsem0, fsem1):
        wid = lax.axis_index("s") * NC + lax.axis_index("c")
        nchunks = jnp.where(wid < 4, jnp.int32(base_chunks + 1),
                            jnp.int32(base_chunks))
        dump = B + wid
        sems = [sem0, sem1, sem2, sem3, sem4, sem5, sem6, sem7]
        iot = lax.iota(jnp.int32, L)

        def chunk_col(j):
            return (j * NW + wid) * W

        def start_slab(j, slot):
            pltpu.async_copy(
                table_hbm.at[:, pl.ds(chunk_col(j), W)],
                slab_v.at[slot],
                sems[slot],
            )

        def wait_slab(slot):
            pltpu.make_async_copy(
                table_hbm.at[:, pl.ds(0, W)], slab_v.at[slot], sems[slot]
            ).wait()

        for j0 in range(NBUF):
            start_slab(j0, j0)
        pltpu.sync_copy(idx_hbm, idx_v)

        def init_scatpos(p):
            dv = jnp.full((L,), dump, jnp.int32)
            for q in range(G // L):
                scatpos_v[p, pl.ds(q * L, L)] = dv

        init_scatpos(0)
        init_scatpos(1)
        nsm[0] = jnp.int32(0)
        rsm[0] = jnp.int32(0)
        fsm[0] = jnp.int32(0)

        @pl.loop(0, B // L)
        def _l1(i):
            v = idx_v[pl.ds(i * L, L)]
            cid = lax.shift_right_logical(v, 7)
            m = (cid & (NW - 1)) == wid
            kcnt = jnp.sum(m.astype(jnp.int32))
            pos = jnp.full((L,), i * L, jnp.int32) + iot
            plsc.store_compressed(extp_v.at[...], pos, mask=m)
            packed = extp_v[...]
            n = nsm[0]
            plsc.store_scatter(
                hitpos_v.at[...], [jnp.full((L,), n, jnp.int32) + iot],
                packed, mask=iot < kcnt,
            )
            nsm[0] = n + kcnt

        def flush():
            fcount = fsm[0]
            par = fcount & 1

            def issue(p, fsem, other):
                pltpu.async_copy(
                    rowbuf_v.at[p], out_hbm.at[scatpos_v.at[p]], fsem
                )

                @pl.when(fcount >= 1)
                def _():
                    pltpu.make_async_copy(
                        rowbuf_v.at[1 - p], out_hbm.at[pl.ds(0, G)], other
                    ).wait()
                    init_scatpos(1 - p)

            @pl.when(par == 0)
            def _():
                issue(0, fsem0, fsem1)

            @pl.when(par == 1)
            def _():
                issue(1, fsem1, fsem0)

            fsm[0] = fcount + 1

        def process(list_ref, cnt, slab, slab_base, filt_lo, filt_hi):
            ngrp = (cnt + (L - 1)) // L

            @pl.loop(0, ngrp)
            def _grp(g):
                valid = (jnp.full((L,), g * L, jnp.int32) + iot) < cnt
                pv = list_ref[pl.ds(g * L, L)]
                cols = plsc.load_gather(idx_v.at[...], [pv], mask=valid)
                m = valid & (cols >= filt_lo) & (cols < filt_hi)
                k2 = jnp.sum(m.astype(jnp.int32))
                plsc.store_compressed(extc_v.at[...], cols - slab_base, mask=m)
                plsc.store_compressed(extp_v.at[...], pv, mask=m)

                @pl.loop(0, k2)
                def _entry(e):
                    sel = jnp.full((L,), e, jnp.int32)
                    cloc = plsc.load_gather(extc_v.at[...], [sel])
                    pvec = plsc.load_gather(extp_v.at[...], [sel])
                    r2 = rsm[0]
                    parw = jnp.full((L,), fsm[0] & 1, jnp.int32)
                    rfull = jnp.full((L,), r2, jnp.int32)
                    for q in range(D // L):
                        vals = plsc.load_gather(slab.at[...],
                                                [iot + q * L, cloc])
                        plsc.store_scatter(
                            rowbuf_v.at[...], [parw, rfull, iot + q * L], vals
                        )
                    plsc.store_scatter(
                        scatpos_v.at[...], [parw, rfull], pvec, mask=iot == 0
                    )
                    full = r2 + 1 == G

                    @pl.when(full)
                    def _():
                        flush()

                    rsm[0] = jnp.where(full, jnp.int32(0), r2 + 1)

        nsup = (base_chunks + 1 + (SUP - 1)) // SUP

        @pl.loop(0, nsup)
        def _sup(s):
            n = nsm[0]
            n2sm[0] = jnp.int32(0)
            ngrp = (n + (L - 1)) // L

            @pl.loop(0, ngrp)
            def _l15(g):
                valid = (jnp.full((L,), g * L, jnp.int32) + iot) < n
                pv = hitpos_v[pl.ds(g * L, L)]
                cols = plsc.load_gather(idx_v.at[...], [pv], mask=valid)
                jv = lax.shift_right_logical(
                    lax.shift_right_logical(cols, 7) - wid, 5)
                m = valid & (jv >= s * SUP) & (jv < (s + 1) * SUP)
                kcnt = jnp.sum(m.astype(jnp.int32))
                plsc.store_compressed(extp_v.at[...], pv, mask=m)
                packed = extp_v[...]
                n2 = n2sm[0]
                plsc.store_scatter(
                    sh_v.at[...], [jnp.full((L,), n2, jnp.int32) + iot],
                    packed, mask=iot < kcnt,
                )
                n2sm[0] = n2 + kcnt

            n2 = n2sm[0]
            for t in range(SUP):
                j = s * SUP + t
                slot = t % NBUF

                @pl.when(j < nchunks)
                def _():
                    wait_slab(slot)
                    c0 = chunk_col(j)
                    process(sh_v, n2, slab_v.at[slot], c0, c0, c0 + W)

                    @pl.when(j + NBUF < nchunks)
                    def _():
                        start_slab(j + NBUF, slot)

        @pl.when(wid == 4)
        def _():
            pltpu.sync_copy(table_hbm.at[:, pl.ds(TAIL, V - TAIL)], tail_v)
            process(hitpos_v, nsm[0], tail_v, TAIL, TAIL, V)

        flush()
        fcount = fsm[0]
        last = (fcount - 1) & 1

        @pl.when(last == 0)
        def _():
            pltpu.make_async_copy(
                rowbuf_v.at[0], out_hbm.at[pl.ds(0, G)], fsem0
            ).wait()

        @pl.when(last == 1)
        def _():
            pltpu.make_async_copy(
                rowbuf_v.at[1], out_hbm.at[pl.ds(0, G)], fsem1
            ).wait()

    return k(table_t, idx)


def kernel(nodes, ordered_embs):
    idx = nodes.reshape((nodes.shape[0],)).astype(jnp.int32)
    table_t = ordered_embs.T
    out3 = _gather_stream(idx, table_t)
    return out3[: nodes.shape[0], :64]
